# Initial kernel scaffold; baseline (speedup 1.0000x reference)
import jax
import jax.numpy as jnp
from jax.experimental import pallas as pl


def kernel(inputs, table):
    raise NotImplementedError("write your pallas kernel here")


# speedup vs baseline: 2.1054x; 2.1054x over previous
import functools

import jax
import jax.numpy as jnp
from jax import lax
from jax.experimental import pallas as pl
from jax.experimental.pallas import tpu as pltpu
from jax.experimental.pallas import tpu_sc as plsc

_D = 64
_MAXQ = 40
_NC = 2
_NS = 16
_L = 16
_NW = _NC * _NS

_CHUNK = 1024
_XFER = 128


@functools.cache
def _make_lookup(n_idx: int):
    b_per_w = n_idx // _NW
    n_chunks = b_per_w // _CHUNK
    mesh = plsc.VectorSubcoreMesh(core_axis_name="c", subcore_axis_name="s")

    @functools.partial(
        pl.kernel,
        out_type=jax.ShapeDtypeStruct((n_idx, _D), jnp.float32),
        mesh=mesh,
        scratch_types=[
            pltpu.VMEM((_CHUNK,), jnp.int32),
            pltpu.VMEM((_CHUNK, _D), jnp.float32),
            pltpu.SemaphoreType.DMA,
        ],
        compiler_params=pltpu.CompilerParams(use_tc_tiling_on_sc=False),
    )
    def lookup(idx_hbm, table_hbm, out_hbm, idx_v, rows_v, sem):
        wid = lax.axis_index("s") * _NC + lax.axis_index("c")
        base = wid * b_per_w

        def chunk_body(ci, carry):
            off = base + ci * _CHUNK
            pltpu.sync_copy(idx_hbm.at[pl.ds(off, _CHUNK)], idx_v)

            def clip_body(i, c):
                sl = pl.ds(pl.multiple_of(i * _L, _L), _L)
                idx_v[sl] = jnp.clip(idx_v[sl], 0, _MAXQ)
                return c

            lax.fori_loop(0, _CHUNK // _L, clip_body, 0)

            copies = [
                pltpu.async_copy(
                    table_hbm.at[idx_v.at[pl.ds(j * _XFER, _XFER)]],
                    rows_v.at[pl.ds(j * _XFER, _XFER)],
                    sem,
                )
                for j in range(_CHUNK // _XFER)
            ]
            for c in copies:
                c.wait()

            pltpu.sync_copy(rows_v, out_hbm.at[pl.ds(off, _CHUNK)])
            return carry

        lax.fori_loop(0, n_chunks, chunk_body, 0)

    return lookup


def kernel(inputs, table):
    b, s = inputs.shape
    idx = inputs.reshape(-1).astype(jnp.int32)
    out = _make_lookup(idx.shape[0])(idx, table)
    return out.reshape(b, s, _D)


# speedup vs baseline: 2.1102x; 1.0023x over previous
import functools

import jax
import jax.numpy as jnp
from jax import lax
from jax.experimental import pallas as pl
from jax.experimental.pallas import tpu as pltpu
from jax.experimental.pallas import tpu_sc as plsc

_D = 64
_MAXQ = 40
_NC = 2
_NS = 16
_L = 16
_NW = _NC * _NS

_CHUNK = 640
_XFER = 128
_NBUF = 2


@functools.cache
def _make_lookup(n_idx: int):
    b_per_w = n_idx // _NW
    n_chunks = b_per_w // _CHUNK
    assert n_chunks % _NBUF == 0
    mesh = plsc.VectorSubcoreMesh(core_axis_name="c", subcore_axis_name="s")

    @functools.partial(
        pl.kernel,
        out_type=jax.ShapeDtypeStruct((n_idx, _D), jnp.float32),
        mesh=mesh,
        scratch_types=[
            pltpu.VMEM((_NBUF, _CHUNK), jnp.int32),
            pltpu.VMEM((_NBUF, _CHUNK, _D), jnp.float32),
            pltpu.SemaphoreType.DMA,
            pltpu.SemaphoreType.DMA,
            pltpu.SemaphoreType.DMA,
            pltpu.SemaphoreType.DMA,
        ],
        compiler_params=pltpu.CompilerParams(use_tc_tiling_on_sc=False),
    )
    def lookup(idx_hbm, table_hbm, out_hbm, idx_v, rows_v, isem, gsem, os0, os1):
        osems = (os0, os1)
        wid = lax.axis_index("s") * _NC + lax.axis_index("c")
        base = wid * b_per_w

        def idx_copy(ci, slot):
            return pltpu.make_async_copy(
                idx_hbm.at[pl.ds(base + ci * _CHUNK, _CHUNK)], idx_v.at[slot], isem
            )

        def gathers(slot):
            return [
                pltpu.make_async_copy(
                    table_hbm.at[idx_v.at[slot].at[pl.ds(j * _XFER, _XFER)]],
                    rows_v.at[slot].at[pl.ds(j * _XFER, _XFER)],
                    gsem,
                )
                for j in range(_CHUNK // _XFER)
            ]

        def out_copy(ci, slot):
            return pltpu.make_async_copy(
                rows_v.at[slot], out_hbm.at[pl.ds(base + ci * _CHUNK, _CHUNK)],
                osems[slot],
            )

        idx_copy(0, 0).start()

        def pair_body(g, carry):
            for b in range(_NBUF):
                ci = g * _NBUF + b
                idx_copy(ci, b).wait()

                @pl.when(ci + 1 < n_chunks)
                def _():
                    idx_copy(ci + 1, (b + 1) % _NBUF).start()

                for i in range(_CHUNK // _L):
                    sl = pl.ds(i * _L, _L)
                    idx_v[b, sl] = jnp.clip(idx_v[b, sl], 0, _MAXQ)

                @pl.when(ci >= _NBUF)
                def _():
                    out_copy(ci - _NBUF, b).wait()

                gs = gathers(b)
                for gcp in gs:
                    gcp.start()
                for gcp in gs:
                    gcp.wait()

                out_copy(ci, b).start()
            return carry

        lax.fori_loop(0, n_chunks // _NBUF, pair_body, 0)
        for b in range(_NBUF):
            out_copy(n_chunks - _NBUF + b, b).wait()

    return lookup


def kernel(inputs, table):
    b, s = inputs.shape
    idx = inputs.reshape(-1).astype(jnp.int32)
    out = _make_lookup(idx.shape[0])(idx, table)
    return out.reshape(b, s, _D)


# speedup vs baseline: 3.8809x; 1.8391x over previous
import functools

import jax
import jax.numpy as jnp
from jax import lax
from jax.experimental import pallas as pl
from jax.experimental.pallas import tpu as pltpu
from jax.experimental.pallas import tpu_sc as plsc

_D = 64
_MAXQ = 40
_NC = 2
_NS = 16
_L = 16
_NW = _NC * _NS

_CHUNK = 640
_NBUF = 2
_GRP = 16


@functools.cache
def _make_lookup(n_idx: int):
    b_per_w = n_idx // _NW
    n_chunks = b_per_w // _CHUNK
    assert n_chunks % _NBUF == 0
    mesh = plsc.VectorSubcoreMesh(core_axis_name="c", subcore_axis_name="s")

    @functools.partial(
        pl.kernel,
        out_type=jax.ShapeDtypeStruct((n_idx, _D), jnp.float32),
        mesh=mesh,
        scratch_types=[
            pltpu.VMEM((_NBUF, _CHUNK), jnp.int32),
            pltpu.VMEM((_NBUF, _CHUNK, _D), jnp.float32),
            pltpu.VMEM((_MAXQ + 2, _D), jnp.float32),
            pltpu.SemaphoreType.DMA,
            pltpu.SemaphoreType.DMA,
            pltpu.SemaphoreType.DMA,
        ],
        compiler_params=pltpu.CompilerParams(
            use_tc_tiling_on_sc=False, needs_layout_passes=False
        ),
    )
    def lookup(idx_hbm, table_hbm, out_hbm, idx_v, rows_v, tab_v, isem, os0, os1):
        osems = (os0, os1)
        wid = lax.axis_index("s") * _NC + lax.axis_index("c")
        base = wid * b_per_w

        pltpu.sync_copy(table_hbm, tab_v)

        def idx_copy(ci, slot):
            return pltpu.make_async_copy(
                idx_hbm.at[pl.ds(base + ci * _CHUNK, _CHUNK)], idx_v.at[slot], isem
            )

        def out_copy(ci, slot):
            return pltpu.make_async_copy(
                rows_v.at[slot], out_hbm.at[pl.ds(base + ci * _CHUNK, _CHUNK)],
                osems[slot],
            )

        cols = [jax.lax.iota(jnp.int32, _L) + c * _L for c in range(_D // _L)]

        idx_copy(0, 0).start()

        def pair_body(g, carry):
            for b in range(_NBUF):
                ci = g * _NBUF + b
                idx_copy(ci, b).wait()

                @pl.when(ci + 1 < n_chunks)
                def _():
                    idx_copy(ci + 1, (b + 1) % _NBUF).start()

                @pl.when(ci >= _NBUF)
                def _():
                    out_copy(ci - _NBUF, b).wait()

                def grp_body(gr, c2):
                    r0 = gr * _GRP
                    vidx = jnp.clip(idx_v[b, pl.ds(r0, _GRP)], 0, _MAXQ)
                    for rr in range(_GRP):
                        row = jnp.full((_L,), vidx[rr], dtype=jnp.int32)
                        for c in range(_D // _L):
                            vals = plsc.load_gather(tab_v, [row, cols[c]])
                            rows_v[b, r0 + rr, pl.ds(c * _L, _L)] = vals
                    return c2

                lax.fori_loop(0, _CHUNK // _GRP, grp_body, 0)

                out_copy(ci, b).start()
            return carry

        lax.fori_loop(0, n_chunks // _NBUF, pair_body, 0)
        for b in range(_NBUF):
            out_copy(n_chunks - _NBUF + b, b).wait()

    return lookup


def kernel(inputs, table):
    b, s = inputs.shape
    idx = inputs.reshape(-1).astype(jnp.int32)
    out = _make_lookup(idx.shape[0])(idx, table)
    return out.reshape(b, s, _D)


# speedup vs baseline: 5.7763x; 1.4884x over previous
import functools

import jax
import jax.numpy as jnp
from jax import lax
from jax.experimental import pallas as pl
from jax.experimental.pallas import tpu as pltpu
from jax.experimental.pallas import tpu_sc as plsc

_D = 64
_MAXQ = 40
_NC = 2
_NS = 16
_L = 16
_NW = _NC * _NS

_CHUNK = 640
_XFER = 640
_NBUF = 2


@functools.cache
def _make_lookup(n_idx: int):
    b_per_w = n_idx // _NW
    n_chunks = b_per_w // _CHUNK
    assert n_chunks % _NBUF == 0
    mesh = plsc.VectorSubcoreMesh(core_axis_name="c", subcore_axis_name="s")

    @functools.partial(
        pl.kernel,
        out_type=jax.ShapeDtypeStruct((n_idx, _D), jnp.float32),
        mesh=mesh,
        scratch_types=[
            pltpu.VMEM((_NBUF, _CHUNK), jnp.int32),
            pltpu.VMEM((_NBUF, _CHUNK, _D), jnp.float32),
            pltpu.VMEM_SHARED((_MAXQ + 2, _D), jnp.float32),
            pltpu.SemaphoreType.DMA,
            pltpu.SemaphoreType.DMA,
            pltpu.SemaphoreType.DMA,
            pltpu.SemaphoreType.DMA,
        ],
        compiler_params=pltpu.CompilerParams(use_tc_tiling_on_sc=False),
    )
    def lookup(idx_hbm, table_hbm, out_hbm, idx_v, rows_v, tab_v, isem, gsem, os0, os1):
        osems = (os0, os1)
        wid = lax.axis_index("s") * _NC + lax.axis_index("c")
        base = wid * b_per_w

        @pl.when(lax.axis_index("s") == 0)
        def _():
            pltpu.sync_copy(table_hbm, tab_v)

        plsc.subcore_barrier()

        def idx_copy(ci, slot):
            return pltpu.make_async_copy(
                idx_hbm.at[pl.ds(base + ci * _CHUNK, _CHUNK)], idx_v.at[slot], isem
            )

        def gathers(slot):
            return [
                pltpu.make_async_copy(
                    tab_v.at[idx_v.at[slot].at[pl.ds(j * _XFER, _XFER)]],
                    rows_v.at[slot].at[pl.ds(j * _XFER, _XFER)],
                    gsem,
                )
                for j in range(_CHUNK // _XFER)
            ]

        def out_copy(ci, slot):
            return pltpu.make_async_copy(
                rows_v.at[slot], out_hbm.at[pl.ds(base + ci * _CHUNK, _CHUNK)],
                osems[slot],
            )

        idx_copy(0, 0).start()

        def pair_body(g, carry):
            for b in range(_NBUF):
                ci = g * _NBUF + b
                idx_copy(ci, b).wait()

                @pl.when(ci + 1 < n_chunks)
                def _():
                    idx_copy(ci + 1, (b + 1) % _NBUF).start()

                for i in range(_CHUNK // _L):
                    sl = pl.ds(i * _L, _L)
                    idx_v[b, sl] = jnp.clip(idx_v[b, sl], 0, _MAXQ)

                @pl.when(ci >= _NBUF)
                def _():
                    out_copy(ci - _NBUF, b).wait()

                gs = gathers(b)
                for gcp in gs:
                    gcp.start()
                for gcp in gs:
                    gcp.wait()

                out_copy(ci, b).start()
            return carry

        lax.fori_loop(0, n_chunks // _NBUF, pair_body, 0)
        for b in range(_NBUF):
            out_copy(n_chunks - _NBUF + b, b).wait()

    return lookup


def kernel(inputs, table):
    b, s = inputs.shape
    idx = inputs.reshape(-1).astype(jnp.int32)
    out = _make_lookup(idx.shape[0])(idx, table)
    return out.reshape(b, s, _D)
